# baseline (device time: 38030 ns/iter reference)
import numpy as np
import jax
import jax.numpy as jnp
from jax import lax
from jax.experimental import pallas as pl
from jax.experimental.pallas import tpu as pltpu

N_DEV = 4


def _rope_tables(Sq, Dh, n_heads, Bl):
    inv = 1.0 / (10000.0 ** (np.arange(0, Dh, 2) / Dh))
    pos = np.arange(Sq)[:, None] * inv[None, :]
    cos = np.repeat(np.cos(pos), 2, axis=-1).astype(np.float32)
    sin = np.repeat(np.sin(pos), 2, axis=-1).astype(np.float32)
    cos_t = np.tile(np.tile(cos, (1, n_heads)), (Bl, 1))
    sin_t = np.tile(np.tile(sin, (1, n_heads)), (Bl, 1))
    n = n_heads * Dh
    P = np.zeros((n, n), np.float32)
    for h in range(n_heads):
        o = h * Dh
        for k in range(Dh // 2):
            P[o + 2 * k + 1, o + 2 * k] = -1.0
            P[o + 2 * k, o + 2 * k + 1] = 1.0
    return cos_t, sin_t, P


def kernel(x, Wq, Wk, Wv, Wo):
    Bl, Sq, D = x.shape
    HD = Wq.shape[1]
    Dh = 64
    HW = HD // 2
    BS = Bl * Sq
    NH = N_DEV - 1

    wcat = jnp.concatenate([Wq, Wk, Wv, Wo.T], axis=0).astype(jnp.bfloat16)
    cos_np, sin_np, P_np = _rope_tables(Sq, Dh, HD // Dh, Bl)
    cos_t = jnp.asarray(cos_np)
    sin_t = jnp.asarray(sin_np)
    P_m = jnp.asarray(P_np).astype(jnp.bfloat16)

    def body(x_ref, wcat_ref, cos_ref, sin_ref, p_ref, out_ref,
             wcw_ref, wccw_ref, ctx_ref,
             cw_send, cw_recv, ccw_send, ccw_recv):
        me = lax.axis_index("i")
        left = lax.rem(me + N_DEV - 1, N_DEV)
        right = lax.rem(me + 1, N_DEV)

        barrier = pltpu.get_barrier_semaphore()
        for nbr in (left, right):
            pl.semaphore_signal(
                barrier, inc=1,
                device_id=(nbr,), device_id_type=pl.DeviceIdType.MESH,
            )
        pl.semaphore_wait(barrier, 2)

        RC = 2 * D

        def make_chunk(u, c, ccw):
            w_ref = wccw_ref if ccw else wcw_ref
            lo = c * RC
            if u == 0:
                cols = (HW, HD) if ccw else (0, HW)
                src = wcat_ref.at[lo:lo + RC, cols[0]:cols[1]]
            else:
                src = w_ref.at[u - 1, lo:lo + RC, :]
            sems = (ccw_send, ccw_recv) if ccw else (cw_send, cw_recv)
            tgt = left if ccw else right
            r = pltpu.make_async_remote_copy(
                src_ref=src, dst_ref=w_ref.at[u, lo:lo + RC, :],
                send_sem=sems[0].at[u, c], recv_sem=sems[1].at[u, c],
                device_id=(tgt,), device_id_type=pl.DeviceIdType.MESH,
            )
            r.start()
            return r

        rd = {}
        for c in range(2):
            rd["cw", 0, c] = make_chunk(0, c, ccw=False)
            rd["ccw", 0, c] = make_chunk(0, c, ccw=True)

        x2 = x_ref[...].reshape(BS, D).astype(jnp.bfloat16)
        cos2 = cos_ref[...]
        sin2 = sin_ref[...]
        pm = p_ref[...]

        def attn_weights(wqk):
            width = wqk.shape[1]
            cw_ = cos2[:, :width]
            sw_ = sin2[:, :width]
            pw_ = pm[:width, :width]
            q = jnp.dot(x2, wqk[0:D], preferred_element_type=jnp.float32)
            k = jnp.dot(x2, wqk[D:2 * D], preferred_element_type=jnp.float32)
            qp = jnp.dot(q.astype(jnp.bfloat16), pw_, preferred_element_type=jnp.float32)
            kp = jnp.dot(k.astype(jnp.bfloat16), pw_, preferred_element_type=jnp.float32)
            q = (q * cw_ + qp * sw_).astype(jnp.bfloat16)
            k = (k * cw_ + kp * sw_).astype(jnp.bfloat16)
            ws = {}
            for b in range(Bl):
                for h in range(width // Dh):
                    qh = q[b * Sq:(b + 1) * Sq, h * Dh:(h + 1) * Dh]
                    kh = k[b * Sq:(b + 1) * Sq, h * Dh:(h + 1) * Dh]
                    s = lax.dot_general(
                        qh, kh, (((1,), (1,)), ((), ())),
                        preferred_element_type=jnp.float32,
                    ) * 0.125
                    w = jnp.exp(s)
                    denom = jnp.sum(w, axis=-1, keepdims=True)
                    ws[b, h] = (w.astype(jnp.bfloat16), denom)
            return ws

        def attn_out(wvo, ws):
            width = wvo.shape[1]
            v = jnp.dot(
                x2, wvo[0:D], preferred_element_type=jnp.float32
            ).astype(jnp.bfloat16)
            for b in range(Bl):
                for h in range(width // Dh):
                    vh = v[b * Sq:(b + 1) * Sq, h * Dh:(h + 1) * Dh]
                    w, denom = ws[b, h]
                    cx = jnp.dot(
                        w, vh, preferred_element_type=jnp.float32
                    ) / denom
                    ctx_ref[b * Sq:(b + 1) * Sq, h * Dh:(h + 1) * Dh] = (
                        cx.astype(jnp.bfloat16)
                    )
            return lax.dot_general(
                ctx_ref[:, :width], wvo[D:2 * D],
                (((1,), (1,)), ((), ())),
                preferred_element_type=jnp.float32,
            )

        def compute_block(wblk):
            return attn_out(wblk[2 * D:4 * D], attn_weights(wblk[0:2 * D]))

        acc = compute_block(wcat_ref[...])

        for c in range(2):
            rd["cw", 0, c].wait_recv()
            rd["cw", 1, c] = make_chunk(1, c, ccw=False)
            rd["ccw", 0, c].wait_recv()
            rd["ccw", 1, c] = make_chunk(1, c, ccw=True)
        acc = acc + compute_block(wcw_ref[0])
        acc = acc + compute_block(wccw_ref[0])

        for c in range(2):
            rd["cw", 1, c].wait_recv()
            rd["cw", 2, c] = make_chunk(2, c, ccw=False)
            rd["ccw", 1, c].wait_recv()
            rd["ccw", 2, c] = make_chunk(2, c, ccw=True)
        acc = acc + compute_block(wcw_ref[1])
        acc = acc + compute_block(wccw_ref[1])

        rd["cw", 2, 0].wait_recv()
        ws_cw = attn_weights(wcw_ref[2, 0:2 * D])
        rd["ccw", 2, 0].wait_recv()
        ws_ccw = attn_weights(wccw_ref[2, 0:2 * D])
        rd["cw", 2, 1].wait_recv()
        acc = acc + attn_out(wcw_ref[2, 2 * D:4 * D], ws_cw)
        rd["ccw", 2, 1].wait_recv()
        acc = acc + attn_out(wccw_ref[2, 2 * D:4 * D], ws_ccw)

        for key in rd:
            rd[key].wait_send()

        out_ref[...] = acc.reshape(Bl, Sq, D)

    return pl.pallas_call(
        body,
        out_shape=jax.ShapeDtypeStruct((Bl, Sq, D), jnp.float32),
        in_specs=[pl.BlockSpec(memory_space=pltpu.VMEM)] * 5,
        out_specs=pl.BlockSpec(memory_space=pltpu.VMEM),
        scratch_shapes=[
            pltpu.VMEM((NH, 4 * D, HW), jnp.bfloat16),
            pltpu.VMEM((NH, 4 * D, HW), jnp.bfloat16),
            pltpu.VMEM((BS, HD), jnp.bfloat16),
            pltpu.SemaphoreType.DMA((NH, 2)),
            pltpu.SemaphoreType.DMA((NH, 2)),
            pltpu.SemaphoreType.DMA((NH, 2)),
            pltpu.SemaphoreType.DMA((NH, 2)),
        ],
        compiler_params=pltpu.CompilerParams(collective_id=0),
    )(x, wcat, cos_t, sin_t, P_m)
